# Initial kernel scaffold; baseline (speedup 1.0000x reference)
import jax
import jax.numpy as jnp
from jax.experimental import pallas as pl


def kernel(x, gate_w, gate_b, W1, W2, W3, sw1, sw2, sw3):
    raise NotImplementedError("write your pallas kernel here")


# speedup vs baseline: 2.9210x; 2.9210x over previous
import jax
import jax.numpy as jnp
from jax.experimental import pallas as pl

T = 2048
D = 768
E = 16
FM = 256
G = 8
KG = 4
SCALE = 2.5
TT = 256

_DOT_PREC = jax.lax.Precision.DEFAULT


def _dot(a, b):
    return jax.lax.dot_general(a, b, (((1,), (1,)), ((), ())),
                               precision=_DOT_PREC,
                               preferred_element_type=jnp.float32)


def _moe_kernel(x_ref, gate_w_ref, gate_b_ref, w1_ref, w2_ref, w3_ref,
                sw1_ref, sw2_ref, sw3_ref, out_ref):
    x = x_ref[...]

    scores = jax.nn.sigmoid(_dot(x, gate_w_ref[...]))
    sb = scores + gate_b_ref[...]
    gs = sb.reshape(TT, G, 2).sum(axis=-1)
    ga = gs[:, :, None]
    gb = gs[:, None, :]
    gidx = jax.lax.broadcasted_iota(jnp.int32, (TT, G, G), 1)
    oidx = jax.lax.broadcasted_iota(jnp.int32, (TT, G, G), 2)
    beats = jnp.logical_or(gb > ga, jnp.logical_and(gb == ga, oidx < gidx))
    rank = jnp.where(beats, 1.0, 0.0).sum(axis=-1)
    sel_g = jnp.where(rank < KG, 1.0, 0.0)
    sel_e = jnp.broadcast_to(sel_g[:, :, None], (TT, G, 2)).reshape(TT, E)
    w = sel_e * scores
    cw = w * (SCALE / w.sum(axis=-1, keepdims=True))

    hs = jax.nn.silu(_dot(x, sw1_ref[...])) * _dot(x, sw3_ref[...])
    acc = _dot(hs, sw2_ref[...])

    for e in range(E):
        h1 = _dot(x, w1_ref[e])
        h3 = _dot(x, w3_ref[e])
        h = jax.nn.silu(h1) * h3 * cw[:, e:e + 1]
        acc += _dot(h, w2_ref[e])

    out_ref[...] = acc


@jax.jit
def kernel(x, gate_w, gate_b, W1, W2, W3, sw1, sw2, sw3):
    grid = (T // TT,)
    return pl.pallas_call(
        _moe_kernel,
        grid=grid,
        in_specs=[
            pl.BlockSpec((TT, D), lambda t: (t, 0)),
            pl.BlockSpec((E, D), lambda t: (0, 0)),
            pl.BlockSpec((1, E), lambda t: (0, 0)),
            pl.BlockSpec((E, FM, D), lambda t: (0, 0, 0)),
            pl.BlockSpec((E, D, FM), lambda t: (0, 0, 0)),
            pl.BlockSpec((E, FM, D), lambda t: (0, 0, 0)),
            pl.BlockSpec((FM, D), lambda t: (0, 0)),
            pl.BlockSpec((D, FM), lambda t: (0, 0)),
            pl.BlockSpec((FM, D), lambda t: (0, 0)),
        ],
        out_specs=pl.BlockSpec((TT, D), lambda t: (t, 0)),
        out_shape=jax.ShapeDtypeStruct((T, D), x.dtype),
    )(x, gate_w, gate_b.reshape(1, E), W1, W2, W3, sw1, sw2, sw3)
